# baseline (speedup 1.0000x reference)
import jax
import jax.numpy as jnp
from jax.experimental import pallas as pl
from jax.experimental.pallas import tpu as pltpu


_CHUNK_ROWS = 512
_N_SLOTS = 8
_DEPTH = 4


def _memcpy_kernel(x_ref, o_ref, buf, in_sems, out_sems):
    n_chunks = x_ref.shape[0] // _CHUNK_ROWS

    def in_copy(i):
        s = i % _N_SLOTS
        return pltpu.make_async_copy(
            x_ref.at[pl.ds(i * _CHUNK_ROWS, _CHUNK_ROWS)], buf.at[s], in_sems.at[s]
        )

    def out_copy(i):
        s = i % _N_SLOTS
        return pltpu.make_async_copy(
            buf.at[s], o_ref.at[pl.ds(i * _CHUNK_ROWS, _CHUNK_ROWS)], out_sems.at[s]
        )

    for i in range(n_chunks):
        if i >= _N_SLOTS:
            out_copy(i - _N_SLOTS).wait()
        in_copy(i).start()
        j = i - _DEPTH
        if j >= 0:
            in_copy(j).wait()
            out_copy(j).start()
    for j in range(max(0, n_chunks - _DEPTH), n_chunks):
        in_copy(j).wait()
        out_copy(j).start()
    for j in range(max(0, n_chunks - _N_SLOTS), n_chunks):
        out_copy(j).wait()


def kernel(inputs, memories):
    del memories
    B, T, d = inputs.shape
    x = inputs.reshape(B * T, d)
    out = pl.pallas_call(
        _memcpy_kernel,
        out_shape=jax.ShapeDtypeStruct(x.shape, x.dtype),
        in_specs=[pl.BlockSpec(memory_space=pl.ANY)],
        out_specs=pl.BlockSpec(memory_space=pl.ANY),
        scratch_shapes=[
            pltpu.VMEM((_N_SLOTS, _CHUNK_ROWS, d), jnp.float32),
            pltpu.SemaphoreType.DMA((_N_SLOTS,)),
            pltpu.SemaphoreType.DMA((_N_SLOTS,)),
        ],
    )(x)
    return out.reshape(B, T, d)
